# Initial kernel scaffold; baseline (speedup 1.0000x reference)
import jax
import jax.numpy as jnp
from jax.experimental import pallas as pl


def kernel(x, edge_index, W1, b1, a, Wd, bd):
    raise NotImplementedError("write your pallas kernel here")


# speedup vs baseline: 10.7078x; 10.7078x over previous
import functools

import jax
import jax.numpy as jnp
from jax import lax
from jax.experimental import pallas as pl
from jax.experimental.pallas import tpu as pltpu
from jax.experimental.pallas import tpu_sc as plsc

N = 10000
F_IN = 128
HID = 512
E = 320000
DROP_FEAT = 0.2

NC, NS, LANES = 2, 16, 16
BATCH = 128
ESL = E + N
TB = 168
SLOTS = NS * TB * BATCH
PAD = SLOTS - ESL
NPAD = 10240
RT = NPAD // NS
BN = 1000
NBLK = N // BN

_f32 = jnp.float32
_i32 = jnp.int32


def _prep_body(colpad, rowpad, perm_t, deg_out, row2_out,
               idx_v, gbuf_v, zeros_v, ones_v, shared_deg, sem):
    c = lax.axis_index("c")
    t = lax.axis_index("s")
    base = t * TB

    @pl.when(c == 0)
    def _():
        @pl.loop(0, RT // LANES)
        def _(i):
            zeros_v[pl.ds(i * LANES, LANES)] = jnp.zeros((LANES,), _f32)

        @pl.loop(0, BATCH // LANES)
        def _(i):
            ones_v[pl.ds(i * LANES, LANES)] = jnp.full((LANES,), 1.0, _f32)

        pltpu.sync_copy(zeros_v, shared_deg.at[pl.ds(t * RT, RT)])
        plsc.subcore_barrier()
        pltpu.sync_copy(colpad.at[pl.ds(base, TB)], idx_v)

        @pl.loop(0, TB)
        def _(j):
            pltpu.sync_copy(ones_v, shared_deg.at[idx_v.at[j]], add=True)

        plsc.subcore_barrier()
        pltpu.sync_copy(shared_deg.at[pl.ds(t * RT, RT)],
                        deg_out.at[pl.ds(t * RT, RT)])

    @pl.when(c == 1)
    def _():
        pltpu.sync_copy(rowpad.at[pl.ds(base, TB)], idx_v)

        @pl.loop(0, TB)
        def _(j):
            pltpu.async_copy(perm_t.at[idx_v.at[j]], gbuf_v.at[j], sem).wait()

        pltpu.sync_copy(gbuf_v, row2_out.at[pl.ds(base, TB)])


_prep = pl.kernel(
    _prep_body,
    out_type=(jax.ShapeDtypeStruct((NPAD,), _f32),
              jax.ShapeDtypeStruct((NS * TB, BATCH), _i32)),
    mesh=plsc.VectorSubcoreMesh(core_axis_name="c", subcore_axis_name="s",
                                num_cores=NC, num_subcores=NS),
    scratch_types=[
        pltpu.VMEM((TB, BATCH), _i32),
        pltpu.VMEM((TB, BATCH), _i32),
        pltpu.VMEM((RT,), _f32),
        pltpu.VMEM((BATCH,), _f32),
        pltpu.VMEM_SHARED((NPAD,), _f32),
        pltpu.SemaphoreType.DMA,
    ],
)


def _gemm_body(x_ref, mask_ref, w1_ref, deg_ref, degp_ref, *out_refs):
    xm = x_ref[...] * mask_ref[...]
    h = jnp.dot(xm, w1_ref[...], preferred_element_type=_f32)
    dinv = lax.rsqrt(deg_ref[...])
    dinvp = lax.rsqrt(degp_ref[...])
    g = h * dinv
    w = h * dinvp
    for s in range(4):
        out_refs[s][...] = g[:, s * 128:(s + 1) * 128]
        out_refs[4 + s][...] = w[:, s * 128:(s + 1) * 128]


def _run_gemm(x, mask, W1, deg2d, degp2d):
    return pl.pallas_call(
        _gemm_body,
        grid=(NBLK,),
        in_specs=[
            pl.BlockSpec((BN, F_IN), lambda i: (i, 0)),
            pl.BlockSpec((1, F_IN), lambda i: (0, 0)),
            pl.BlockSpec((F_IN, HID), lambda i: (0, 0)),
            pl.BlockSpec((BN, 1), lambda i: (i, 0)),
            pl.BlockSpec((BN, 1), lambda i: (i, 0)),
        ],
        out_specs=[pl.BlockSpec((BN, 128), lambda i: (i, 0))] * 8,
        out_shape=[jax.ShapeDtypeStruct((N, 128), _f32)] * 8,
    )(x, mask, W1, deg2d, degp2d)


IDXCH = 24
ZROWS = 64


def _scatter_body(g0, g1, g2, g3, w0, w1, w2, w3, rowpad, row2pad, colpad,
                  acc_out, colidx_v, rowidx_v, zero_v, buf0, buf1,
                  shared_acc, sem0, sem1):
    c = lax.axis_index("c")
    t = lax.axis_index("s")
    base = t * TB
    tabs = ((g0, g1, g2, g3), (w0, w1, w2, w3))

    @pl.loop(0, ZROWS)
    def _(i):
        for k in range(BATCH // LANES):
            zero_v[i, pl.ds(k * LANES, LANES)] = jnp.zeros((LANES,), _f32)

    for v in range(2):
        rowsrc = rowpad if v == 0 else row2pad
        for ci in range(NC):
            @pl.when(c == ci)
            def _(v=v, ci=ci, rowsrc=rowsrc):
                for sl in range(2):
                    s = 2 * ci + sl
                    tab = tabs[v][s]
                    for k in range(RT // ZROWS):
                        pltpu.sync_copy(
                            zero_v,
                            shared_acc.at[pl.ds(t * RT + k * ZROWS, ZROWS)])
                    plsc.subcore_barrier()

                    @pl.loop(0, TB // IDXCH)
                    def _(ch, tab=tab, rowsrc=rowsrc):
                        off = base + ch * IDXCH
                        pltpu.sync_copy(colpad.at[pl.ds(off, IDXCH)],
                                        colidx_v)
                        pltpu.sync_copy(rowsrc.at[pl.ds(off, IDXCH)],
                                        rowidx_v)

                        @pl.loop(0, IDXCH // 2)
                        def _(i, tab=tab):
                            j0 = i * 2
                            d0 = pltpu.async_copy(tab.at[rowidx_v.at[j0]],
                                                  buf0, sem0)
                            d1 = pltpu.async_copy(tab.at[rowidx_v.at[j0 + 1]],
                                                  buf1, sem1)
                            d0.wait()
                            pltpu.sync_copy(buf0,
                                            shared_acc.at[colidx_v.at[j0]],
                                            add=True)
                            d1.wait()
                            pltpu.sync_copy(
                                buf1,
                                shared_acc.at[colidx_v.at[j0 + 1]],
                                add=True)

                    plsc.subcore_barrier()
                    pltpu.sync_copy(shared_acc.at[pl.ds(t * RT, RT)],
                                    acc_out.at[v, s, pl.ds(t * RT, RT)])
                    plsc.subcore_barrier()


_scatter = pl.kernel(
    _scatter_body,
    out_type=jax.ShapeDtypeStruct((2, 4, NPAD, 128), _f32),
    mesh=plsc.VectorSubcoreMesh(core_axis_name="c", subcore_axis_name="s",
                                num_cores=NC, num_subcores=NS),
    scratch_types=[
        pltpu.VMEM((IDXCH, BATCH), _i32),
        pltpu.VMEM((IDXCH, BATCH), _i32),
        pltpu.VMEM((ZROWS, 128), _f32),
        pltpu.VMEM((BATCH, 128), _f32),
        pltpu.VMEM((BATCH, 128), _f32),
        pltpu.VMEM_SHARED((NPAD, 128), _f32),
        pltpu.SemaphoreType.DMA,
        pltpu.SemaphoreType.DMA,
    ],
)


def _epi_body(deg_ref, b1_ref, wd_ref, bd_ref, a_ref, *rest):
    acc_refs = rest[:8]
    pos_ref, neg_ref = rest[8], rest[9]
    dinv = lax.rsqrt(deg_ref[...])
    wsum = jnp.sum(wd_ref[...], axis=0, keepdims=True)
    bdsum = jnp.sum(bd_ref[...])
    a = a_ref[0, 0]
    for v in range(2):
        tot = jnp.zeros((BN, 1), _f32)
        for s in range(4):
            acc = acc_refs[v * 4 + s][0, 0]
            av = acc * dinv + b1_ref[:, s * 128:(s + 1) * 128]
            z = jnp.maximum(av, 0.0) + a * jnp.minimum(av, 0.0)
            tot = tot + jnp.sum(z * wsum[:, s * 128:(s + 1) * 128],
                                axis=1, keepdims=True)
        out = tot + bdsum
        if v == 0:
            pos_ref[...] = out
        else:
            neg_ref[...] = out


def _run_epi(deg2d, b1r, Wd, bdr, ar, acc):
    acc_specs = [
        pl.BlockSpec((1, 1, BN, 128),
                     functools.partial(lambda i, v=v, s=s: (v, s, i, 0)))
        for v in range(2) for s in range(4)
    ]
    return pl.pallas_call(
        _epi_body,
        grid=(NBLK,),
        in_specs=[
            pl.BlockSpec((BN, 1), lambda i: (i, 0)),
            pl.BlockSpec((1, HID), lambda i: (0, 0)),
            pl.BlockSpec((HID, HID), lambda i: (0, 0)),
            pl.BlockSpec((1, HID), lambda i: (0, 0)),
            pl.BlockSpec((1, 1), lambda i: (0, 0)),
        ] + acc_specs,
        out_specs=[pl.BlockSpec((BN, 1), lambda i: (i, 0))] * 2,
        out_shape=[jax.ShapeDtypeStruct((N, 1), _f32)] * 2,
    )(deg2d, b1r, Wd, bdr, ar, *([acc] * 8))


def kernel(x, edge_index, W1, b1, a, Wd, bd):
    key = jax.random.key(42)
    k1, k2 = jax.random.split(key)
    mask = (jax.random.uniform(k1, (1, F_IN)) >= DROP_FEAT).astype(_f32)
    perm = jax.random.permutation(k2, N).astype(_i32)
    invp = jnp.zeros((N,), _i32).at[perm].set(jnp.arange(N, dtype=_i32))

    row, col = edge_index[0], edge_index[1]
    ar_n = jnp.arange(N, dtype=_i32)
    pad_r = (jnp.arange(PAD, dtype=_i32) * 61) % N
    pad_c = N + (jnp.arange(PAD, dtype=_i32) % LANES)
    rowpad = jnp.concatenate([row, ar_n, pad_r]).reshape(NS * TB, BATCH)
    colpad = jnp.concatenate([col, ar_n, pad_c]).reshape(NS * TB, BATCH)

    deg_pad, row2pad = _prep(colpad, rowpad, perm)
    deg2d = deg_pad[:N].reshape(N, 1)
    degp2d = jnp.take(deg_pad[:N], invp).reshape(N, 1)

    tabs = _run_gemm(x, mask, W1, deg2d, degp2d)
    acc = _scatter(*tabs, rowpad, row2pad, colpad)
    pos2d, neg2d = _run_epi(deg2d, b1.reshape(1, HID), Wd,
                            bd.reshape(1, HID), a.reshape(1, 1), acc)
    return pos2d[:, 0], neg2d[:, 0]


# speedup vs baseline: 12.8161x; 1.1969x over previous
import functools

import jax
import jax.numpy as jnp
from jax import lax
from jax.experimental import pallas as pl
from jax.experimental.pallas import tpu as pltpu
from jax.experimental.pallas import tpu_sc as plsc

N = 10000
F_IN = 128
HID = 512
E = 320000
DROP_FEAT = 0.2

NC, NS, LANES = 2, 16, 16
BATCH = 128
ESL = E + N
TB = 168
SLOTS = NS * TB * BATCH
PAD = SLOTS - ESL
NPAD = 10240
RT = NPAD // NS
BN = 1000
NBLK = N // BN

_f32 = jnp.float32
_i32 = jnp.int32


def _prep_body(colpad, rowpad, perm_t, deg_out, row2_out,
               idx_v, gbuf_v, zeros_v, ones_v, shared_deg, *sems):
    c = lax.axis_index("c")
    t = lax.axis_index("s")
    base = t * TB

    @pl.when(c == 0)
    def _():
        @pl.loop(0, RT // LANES)
        def _(i):
            zeros_v[pl.ds(i * LANES, LANES)] = jnp.zeros((LANES,), _f32)

        @pl.loop(0, BATCH // LANES)
        def _(i):
            ones_v[pl.ds(i * LANES, LANES)] = jnp.full((LANES,), 1.0, _f32)

        pltpu.sync_copy(zeros_v, shared_deg.at[pl.ds(t * RT, RT)])
        plsc.subcore_barrier()
        pltpu.sync_copy(colpad.at[pl.ds(base, TB)], idx_v)

        @pl.loop(0, TB // 8)
        def _(i):
            j0 = i * 8
            ds = [pltpu.async_copy(ones_v, shared_deg.at[idx_v.at[j0 + k]],
                                   sems[k % 4], add=True) for k in range(8)]
            for d in ds:
                d.wait()

        plsc.subcore_barrier()
        pltpu.sync_copy(shared_deg.at[pl.ds(t * RT, RT)],
                        deg_out.at[pl.ds(t * RT, RT)])

    @pl.when(c == 1)
    def _():
        pltpu.sync_copy(rowpad.at[pl.ds(base, TB)], idx_v)

        @pl.loop(0, TB // 8)
        def _(i):
            j0 = i * 8
            ds = [pltpu.async_copy(perm_t.at[idx_v.at[j0 + k]],
                                   gbuf_v.at[j0 + k], sems[k % 4])
                  for k in range(8)]
            for d in ds:
                d.wait()

        pltpu.sync_copy(gbuf_v, row2_out.at[pl.ds(base, TB)])


_prep = pl.kernel(
    _prep_body,
    out_type=(jax.ShapeDtypeStruct((NPAD,), _f32),
              jax.ShapeDtypeStruct((NS * TB, BATCH), _i32)),
    mesh=plsc.VectorSubcoreMesh(core_axis_name="c", subcore_axis_name="s",
                                num_cores=NC, num_subcores=NS),
    scratch_types=[
        pltpu.VMEM((TB, BATCH), _i32),
        pltpu.VMEM((TB, BATCH), _i32),
        pltpu.VMEM((RT,), _f32),
        pltpu.VMEM((BATCH,), _f32),
        pltpu.VMEM_SHARED((NPAD,), _f32),
        pltpu.SemaphoreType.DMA,
        pltpu.SemaphoreType.DMA,
        pltpu.SemaphoreType.DMA,
        pltpu.SemaphoreType.DMA,
    ],
)


def _gemm_body(x_ref, mask_ref, w1_ref, deg_ref, degp_ref, *out_refs):
    xm = x_ref[...] * mask_ref[...]
    h = jnp.dot(xm, w1_ref[...], preferred_element_type=_f32)
    dinv = lax.rsqrt(deg_ref[...])
    dinvp = lax.rsqrt(degp_ref[...])
    g = h * dinv
    w = h * dinvp
    for s in range(4):
        out_refs[s][...] = g[:, s * 128:(s + 1) * 128]
        out_refs[4 + s][...] = w[:, s * 128:(s + 1) * 128]


def _run_gemm(x, mask, W1, deg2d, degp2d):
    return pl.pallas_call(
        _gemm_body,
        grid=(NBLK,),
        in_specs=[
            pl.BlockSpec((BN, F_IN), lambda i: (i, 0)),
            pl.BlockSpec((1, F_IN), lambda i: (0, 0)),
            pl.BlockSpec((F_IN, HID), lambda i: (0, 0)),
            pl.BlockSpec((BN, 1), lambda i: (i, 0)),
            pl.BlockSpec((BN, 1), lambda i: (i, 0)),
        ],
        out_specs=[pl.BlockSpec((BN, 128), lambda i: (i, 0))] * 8,
        out_shape=[jax.ShapeDtypeStruct((N, 128), _f32)] * 8,
    )(x, mask, W1, deg2d, degp2d)


IDXCH = 24
ZROWS = 64


def _scatter_body(g0, g1, g2, g3, w0, w1, w2, w3, rowpad, row2pad, colpad,
                  acc_out, colidx_v, rowidx_v, zero_v, buf0, buf1,
                  shared_acc, gsem0, gsem1, ssem0, ssem1):
    c = lax.axis_index("c")
    t = lax.axis_index("s")
    base = t * TB
    tabs = ((g0, g1, g2, g3), (w0, w1, w2, w3))

    @pl.loop(0, ZROWS)
    def _(i):
        for k in range(BATCH // LANES):
            zero_v[i, pl.ds(k * LANES, LANES)] = jnp.zeros((LANES,), _f32)

    for v in range(2):
        rowsrc = rowpad if v == 0 else row2pad
        for ci in range(NC):
            @pl.when(c == ci)
            def _(v=v, ci=ci, rowsrc=rowsrc):
                for sl in range(2):
                    s = 2 * ci + sl
                    tab = tabs[v][s]
                    for k in range(RT // ZROWS):
                        pltpu.sync_copy(
                            zero_v,
                            shared_acc.at[pl.ds(t * RT + k * ZROWS, ZROWS)])
                    plsc.subcore_barrier()

                    @pl.loop(0, TB // IDXCH)
                    def _(ch, tab=tab, rowsrc=rowsrc):
                        off = base + ch * IDXCH
                        pltpu.sync_copy(colpad.at[pl.ds(off, IDXCH)],
                                        colidx_v)
                        pltpu.sync_copy(rowsrc.at[pl.ds(off, IDXCH)],
                                        rowidx_v)
                        bufs = (buf0, buf1)
                        gsems = (gsem0, gsem1)
                        ssems = (ssem0, ssem1)
                        dg = [None] * IDXCH
                        ds = [None] * IDXCH
                        dg[0] = pltpu.async_copy(tab.at[rowidx_v.at[0]],
                                                 buf0, gsem0)
                        for j in range(IDXCH):
                            b = j % 2
                            dg[j].wait()
                            ds[j] = pltpu.async_copy(
                                bufs[b], shared_acc.at[colidx_v.at[j]],
                                ssems[b], add=True)
                            if j + 1 < IDXCH:
                                if j >= 1:
                                    ds[j - 1].wait()
                                dg[j + 1] = pltpu.async_copy(
                                    tab.at[rowidx_v.at[j + 1]],
                                    bufs[1 - b], gsems[1 - b])
                        ds[IDXCH - 2].wait()
                        ds[IDXCH - 1].wait()

                    plsc.subcore_barrier()
                    pltpu.sync_copy(shared_acc.at[pl.ds(t * RT, RT)],
                                    acc_out.at[v, s, pl.ds(t * RT, RT)])
                    plsc.subcore_barrier()


_scatter = pl.kernel(
    _scatter_body,
    out_type=jax.ShapeDtypeStruct((2, 4, NPAD, 128), _f32),
    mesh=plsc.VectorSubcoreMesh(core_axis_name="c", subcore_axis_name="s",
                                num_cores=NC, num_subcores=NS),
    scratch_types=[
        pltpu.VMEM((IDXCH, BATCH), _i32),
        pltpu.VMEM((IDXCH, BATCH), _i32),
        pltpu.VMEM((ZROWS, 128), _f32),
        pltpu.VMEM((BATCH, 128), _f32),
        pltpu.VMEM((BATCH, 128), _f32),
        pltpu.VMEM_SHARED((NPAD, 128), _f32),
        pltpu.SemaphoreType.DMA,
        pltpu.SemaphoreType.DMA,
        pltpu.SemaphoreType.DMA,
        pltpu.SemaphoreType.DMA,
    ],
)


def _epi_body(deg_ref, b1_ref, wd_ref, bd_ref, a_ref, *rest):
    acc_refs = rest[:8]
    pos_ref, neg_ref = rest[8], rest[9]
    dinv = lax.rsqrt(deg_ref[...])
    wsum = jnp.sum(wd_ref[...], axis=0, keepdims=True)
    bdsum = jnp.sum(bd_ref[...])
    a = a_ref[0, 0]
    for v in range(2):
        tot = jnp.zeros((BN, 1), _f32)
        for s in range(4):
            acc = acc_refs[v * 4 + s][0, 0]
            av = acc * dinv + b1_ref[:, s * 128:(s + 1) * 128]
            z = jnp.maximum(av, 0.0) + a * jnp.minimum(av, 0.0)
            tot = tot + jnp.sum(z * wsum[:, s * 128:(s + 1) * 128],
                                axis=1, keepdims=True)
        out = tot + bdsum
        if v == 0:
            pos_ref[...] = out
        else:
            neg_ref[...] = out


def _run_epi(deg2d, b1r, Wd, bdr, ar, acc):
    acc_specs = [
        pl.BlockSpec((1, 1, BN, 128),
                     functools.partial(lambda i, v=v, s=s: (v, s, i, 0)))
        for v in range(2) for s in range(4)
    ]
    return pl.pallas_call(
        _epi_body,
        grid=(NBLK,),
        in_specs=[
            pl.BlockSpec((BN, 1), lambda i: (i, 0)),
            pl.BlockSpec((1, HID), lambda i: (0, 0)),
            pl.BlockSpec((HID, HID), lambda i: (0, 0)),
            pl.BlockSpec((1, HID), lambda i: (0, 0)),
            pl.BlockSpec((1, 1), lambda i: (0, 0)),
        ] + acc_specs,
        out_specs=[pl.BlockSpec((BN, 1), lambda i: (i, 0))] * 2,
        out_shape=[jax.ShapeDtypeStruct((N, 1), _f32)] * 2,
    )(deg2d, b1r, Wd, bdr, ar, *([acc] * 8))


def kernel(x, edge_index, W1, b1, a, Wd, bd):
    key = jax.random.key(42)
    k1, k2 = jax.random.split(key)
    mask = (jax.random.uniform(k1, (1, F_IN)) >= DROP_FEAT).astype(_f32)
    perm = jax.random.permutation(k2, N).astype(_i32)
    invp = jnp.zeros((N,), _i32).at[perm].set(jnp.arange(N, dtype=_i32))

    row, col = edge_index[0], edge_index[1]
    ar_n = jnp.arange(N, dtype=_i32)
    pad_r = (jnp.arange(PAD, dtype=_i32) * 61) % N
    pad_c = N + (jnp.arange(PAD, dtype=_i32) % LANES)
    rowpad = jnp.concatenate([row, ar_n, pad_r]).reshape(NS * TB, BATCH)
    colpad = jnp.concatenate([col, ar_n, pad_c]).reshape(NS * TB, BATCH)

    deg_pad, row2pad = _prep(colpad, rowpad, perm)
    deg2d = deg_pad[:N].reshape(N, 1)
    degp2d = jnp.take(deg_pad[:N], invp).reshape(N, 1)

    tabs = _run_gemm(x, mask, W1, deg2d, degp2d)
    acc = _scatter(*tabs, rowpad, row2pad, colpad)
    pos2d, neg2d = _run_epi(deg2d, b1.reshape(1, HID), Wd,
                            bd.reshape(1, HID), a.reshape(1, 1), acc)
    return pos2d[:, 0], neg2d[:, 0]
